# baseline (speedup 1.0000x reference)
import functools

import jax
import jax.numpy as jnp
from jax import lax
from jax.experimental import pallas as pl
from jax.experimental.pallas import tpu as pltpu
from jax.experimental.pallas import tpu_sc as plsc

B = 16384
F = 26
D = 16
V = 1000000
NUM_CORES = 2
NUM_SUBCORES = 16
NW = NUM_CORES * NUM_SUBCORES
BPW = B // NW

T1BLK = 131072
T1G = -(-V // T1BLK)
TS_UNITS = T1G * T1BLK

NBUF = 4


def _t1_body(in_ref, out_ref):
    x = in_ref[...]
    x3 = x.reshape(D, T1BLK // 128, 128)
    for q in range(T1BLK // 1024):
        s = jnp.concatenate(
            [x3[:, q * 8 + m, :] for m in range(8)], axis=0)
        out_ref[q * 128:(q + 1) * 128, :] = s.T


_t1 = pl.pallas_call(
    _t1_body,
    grid=(T1G,),
    in_specs=[pl.BlockSpec((D, T1BLK), lambda g: (0, g))],
    out_specs=pl.BlockSpec((T1BLK // 8, 128), lambda g: (g, 0)),
    out_shape=jax.ShapeDtypeStruct((T1G * (T1BLK // 8), 128), jnp.float32),
)


_T2BLK = 16384


def _t2_body(in_ref, out_ref):
    x = in_ref[...]
    for q in range(_T2BLK // 1024):
        w = x[q * 128:(q + 1) * 128, :]
        y = w.T.reshape(8, D, 128)
        for j in range(8):
            out_ref[0, :, q * 1024 + j * 128:q * 1024 + (j + 1) * 128] = y[j]


_t2 = pl.pallas_call(
    _t2_body,
    grid=(F, B // _T2BLK),
    in_specs=[pl.BlockSpec((_T2BLK * D // 128, 128),
                           lambda f, c: (f * (B // _T2BLK) + c, 0))],
    out_specs=pl.BlockSpec((1, D, _T2BLK), lambda f, c: (f, 0, c)),
    out_shape=jax.ShapeDtypeStruct((F, D, B), jnp.float32),
)


def _perm(i):
    return (i & ~1023) | ((i & 127) << 3) | ((i >> 7) & 7)


@functools.partial(
    pl.kernel,
    mesh=plsc.VectorSubcoreMesh(core_axis_name="c", subcore_axis_name="s"),
    out_type=jax.ShapeDtypeStruct((F * B, D), jnp.float32),
    compiler_params=pltpu.CompilerParams(use_tc_tiling_on_sc=False),
    scratch_types=(
        [pltpu.VMEM((F, BPW), jnp.int32)]
        + [pltpu.VMEM((BPW, D), jnp.float32) for _ in range(NBUF)]
        + [pltpu.SemaphoreType.DMA for _ in range(2 * NBUF)]
    ),
)
def _gather_kernel(gidx_hbm, ts_hbm, out_hbm, gidx_v, *bufs_and_sems):
    bufs = bufs_and_sems[:NBUF]
    sem_g = bufs_and_sems[NBUF:2 * NBUF]
    sem_o = bufs_and_sems[2 * NBUF:]
    wid = lax.axis_index("s") * NUM_CORES + lax.axis_index("c")
    b0 = wid * BPW
    pltpu.sync_copy(gidx_hbm.at[:, pl.ds(b0, BPW)], gidx_v)

    def gather(f, slot):
        return pltpu.async_copy(
            ts_hbm.at[gidx_v.at[f]], bufs[slot], sem_g[slot])

    def writeback(f, slot):
        return pltpu.async_copy(
            bufs[slot], out_hbm.at[pl.ds(f * B + b0, BPW)], sem_o[slot])

    gathers = [None] * F
    outs = [None] * F
    for f in range(min(NBUF, F)):
        gathers[f] = gather(f, f % NBUF)
    for f in range(F):
        gathers[f].wait()
        outs[f] = writeback(f, f % NBUF)
        if f + NBUF < F:
            outs[f].wait()
            gathers[f + NBUF] = gather(f + NBUF, f % NBUF)
    for f in range(max(0, F - NBUF), F):
        outs[f].wait()


def kernel(inputs, table):
    ts = _t1(table.T)
    gidx = (_perm(inputs).reshape(16, 8, 128, F)
            .transpose(3, 0, 2, 1).reshape(F, B))
    out5 = _gather_kernel(gidx, ts.reshape(TS_UNITS, D))
    out3 = _t2(out5.reshape(F * B * D // 128, 128))
    return out3.transpose(2, 0, 1)
